# baseline (speedup 1.0000x reference)
import functools

import jax
import jax.numpy as jnp
from jax import lax
from jax.experimental import pallas as pl
from jax.experimental.pallas import tpu as pltpu
from jax.experimental.pallas import tpu_sc as plsc

B, N, C_IN, C_OUT = 2, 4096, 64, 64
K = 15
RADIUS = 2.5
NSAMPLE = 16
KP_EXTENT = 1.2

NSUB = 16
PTS_PER_SUB = N // NSUB
GRP = 8
NGRP = PTS_PER_SUB // GRP
NCHUNK = N // 16
PAD_XYZ = 1000000.0
SC = NSAMPLE * C_IN
CW = 8
SCW = NSAMPLE * CW


_SC_MESH = plsc.VectorSubcoreMesh(core_axis_name="c", subcore_axis_name="s")


def _sc_body(px, py, pz, xt, xit, ctab,
             nidx_o, feat_o, xing_o, coord_o,
             pxv, pyv, pzv, idxbuf, st_nidx,
             gidx0, gidx1, rows_x0, rows_xi0, rows_c0,
             rows_x1, rows_xi1, rows_c1,
             gsx0, gsxi0, gsc0, gsx1, gsxi1, gsc1,
             wsx0, wsxi0, wsc0, wsx1, wsxi1, wsc1):
    b = lax.axis_index("c")
    w = lax.axis_index("s")
    bN = b * N
    pltpu.sync_copy(px.at[pl.ds(bN, N)], pxv)
    pltpu.sync_copy(py.at[pl.ds(bN, N)], pyv)
    pltpu.sync_copy(pz.at[pl.ds(bN, N)], pzv)
    base_local = w * PTS_PER_SUB
    r2 = jnp.float32(RADIUS * RADIUS)
    lanes = lax.iota(jnp.int32, 16)

    def scan_group(g, gidx):
        def point_body(t, carry):
            i_loc = base_local + carry + t
            isplat = jnp.full((16,), i_loc, jnp.int32)
            qx = plsc.load_gather(pxv, [isplat])
            qy = plsc.load_gather(pyv, [isplat])
            qz = plsc.load_gather(pzv, [isplat])

            def scan_cond(st):
                j, cnt = st
                return jnp.logical_and(cnt < NSAMPLE, j < NCHUNK)

            def scan_body(st):
                j, cnt = st
                off = j * 16
                jv = lanes + off
                dx = pxv[pl.ds(off, 16)] - qx
                dy = pyv[pl.ds(off, 16)] - qy
                dz = pzv[pl.ds(off, 16)] - qz
                sq = dx * dx + dy * dy + dz * dz
                m = sq <= r2
                plsc.store_compressed(idxbuf.at[pl.ds(cnt, 16)], jv, mask=m)
                return j + 1, cnt + jnp.sum(m.astype(jnp.int32))

            _, cnt = lax.while_loop(scan_cond, scan_body,
                                    (jnp.int32(0), jnp.int32(0)))
            f = jnp.minimum(cnt, NSAMPLE)
            idx16 = idxbuf[pl.ds(0, 16)]
            first = idx16[0]
            valid = lanes < f
            idxv = jnp.where(valid, idx16, first)
            st_nidx[t, :] = idxv
            gidx[pl.ds(t * 16, 16)] = idxv + bN
            return carry

        lax.fori_loop(0, GRP, point_body, g * GRP)

    def emit_st(g):
        gbase = bN + base_local + g * GRP
        pltpu.sync_copy(st_nidx, nidx_o.at[pl.ds(gbase, GRP)])

    def fslice(g):
        gbase = bN + base_local + g * GRP
        return pl.ds(gbase * 16, GRP * 16)

    def issue_gather(gidx, rx, rxi, rc, sx, sxi, sc):
        pltpu.async_copy(xt.at[gidx], rx, sx)
        pltpu.async_copy(xit.at[gidx], rxi, sxi)
        pltpu.async_copy(ctab.at[gidx], rc, sc)

    def wait_gather(gidx, rx, rxi, rc, sx, sxi, sc):
        pltpu.make_async_copy(xt.at[gidx], rx, sx).wait()
        pltpu.make_async_copy(xit.at[gidx], rxi, sxi).wait()
        pltpu.make_async_copy(ctab.at[gidx], rc, sc).wait()

    def issue_wb(g, rx, rxi, rc, sx, sxi, sc):
        pltpu.async_copy(rx, feat_o.at[fslice(g)], sx)
        pltpu.async_copy(rxi, xing_o.at[fslice(g)], sxi)
        pltpu.async_copy(rc, coord_o.at[fslice(g)], sc)

    def wait_wb(g, rx, rxi, rc, sx, sxi, sc):
        pltpu.make_async_copy(rx, feat_o.at[fslice(g)], sx).wait()
        pltpu.make_async_copy(rxi, xing_o.at[fslice(g)], sxi).wait()
        pltpu.make_async_copy(rc, coord_o.at[fslice(g)], sc).wait()

    scan_group(0, gidx0)
    issue_gather(gidx0, rows_x0, rows_xi0, rows_c0, gsx0, gsxi0, gsc0)
    emit_st(0)
    scan_group(1, gidx1)
    issue_gather(gidx1, rows_x1, rows_xi1, rows_c1, gsx1, gsxi1, gsc1)
    emit_st(1)
    wait_gather(gidx0, rows_x0, rows_xi0, rows_c0, gsx0, gsxi0, gsc0)
    issue_wb(0, rows_x0, rows_xi0, rows_c0, wsx0, wsxi0, wsc0)

    def pair_body(h, _):
        g0 = 2 * h
        g1 = g0 + 1
        scan_group(g0, gidx0)
        wait_wb(g0 - 2, rows_x0, rows_xi0, rows_c0, wsx0, wsxi0, wsc0)
        issue_gather(gidx0, rows_x0, rows_xi0, rows_c0, gsx0, gsxi0, gsc0)
        emit_st(g0)
        wait_gather(gidx1, rows_x1, rows_xi1, rows_c1,
                    gsx1, gsxi1, gsc1)
        issue_wb(g0 - 1, rows_x1, rows_xi1, rows_c1, wsx1, wsxi1, wsc1)
        scan_group(g1, gidx1)
        wait_wb(g0 - 1, rows_x1, rows_xi1, rows_c1, wsx1, wsxi1, wsc1)
        issue_gather(gidx1, rows_x1, rows_xi1, rows_c1, gsx1, gsxi1, gsc1)
        emit_st(g1)
        wait_gather(gidx0, rows_x0, rows_xi0, rows_c0,
                    gsx0, gsxi0, gsc0)
        issue_wb(g0, rows_x0, rows_xi0, rows_c0, wsx0, wsxi0, wsc0)
        return 0

    lax.fori_loop(1, NGRP // 2, pair_body, 0)
    wait_gather(gidx1, rows_x1, rows_xi1, rows_c1, gsx1, gsxi1, gsc1)
    pltpu.sync_copy(rows_x1, feat_o.at[fslice(NGRP - 1)])
    pltpu.sync_copy(rows_xi1, xing_o.at[fslice(NGRP - 1)])
    pltpu.sync_copy(rows_c1, coord_o.at[fslice(NGRP - 1)])
    wait_wb(NGRP - 2, rows_x0, rows_xi0, rows_c0, wsx0, wsxi0, wsc0)


_sc_stage = functools.partial(
    pl.kernel,
    out_type=(
        jax.ShapeDtypeStruct((B * N, NSAMPLE), jnp.int32),
        jax.ShapeDtypeStruct((B * N * NSAMPLE, C_IN), jnp.bfloat16),
        jax.ShapeDtypeStruct((B * N * NSAMPLE, C_IN), jnp.bfloat16),
        jax.ShapeDtypeStruct((B * N * NSAMPLE, CW), jnp.float32),
    ),
    mesh=_SC_MESH,
    compiler_params=pltpu.CompilerParams(
        needs_layout_passes=False,
        use_tc_tiling_on_sc=False,
    ),
    scratch_types=[
        pltpu.VMEM((N,), jnp.float32),
        pltpu.VMEM((N,), jnp.float32),
        pltpu.VMEM((N,), jnp.float32),
        pltpu.VMEM((48,), jnp.int32),
        pltpu.VMEM((GRP, 16), jnp.int32),
        pltpu.VMEM((GRP * 16,), jnp.int32),
        pltpu.VMEM((GRP * 16,), jnp.int32),
        pltpu.VMEM((GRP * 16, C_IN), jnp.bfloat16),
        pltpu.VMEM((GRP * 16, C_IN), jnp.bfloat16),
        pltpu.VMEM((GRP * 16, CW), jnp.float32),
        pltpu.VMEM((GRP * 16, C_IN), jnp.bfloat16),
        pltpu.VMEM((GRP * 16, C_IN), jnp.bfloat16),
        pltpu.VMEM((GRP * 16, CW), jnp.float32),
        pltpu.SemaphoreType.DMA,
        pltpu.SemaphoreType.DMA,
        pltpu.SemaphoreType.DMA,
        pltpu.SemaphoreType.DMA,
        pltpu.SemaphoreType.DMA,
        pltpu.SemaphoreType.DMA,
        pltpu.SemaphoreType.DMA,
        pltpu.SemaphoreType.DMA,
        pltpu.SemaphoreType.DMA,
        pltpu.SemaphoreType.DMA,
        pltpu.SemaphoreType.DMA,
        pltpu.SemaphoreType.DMA,
    ],
)(_sc_body)


RB = 256
NBN = N // RB


def _tc_body(nidx, cg, q8, f2, xi2, wrep_ref, kp_ref,
             out_ref, skip_ref):
    idx = nidx[...]
    s_iota = lax.broadcasted_iota(jnp.int32, (RB, NSAMPLE), 1)
    pad = jnp.logical_and(idx == idx[:, 0:1], s_iota > 0)
    keep = jnp.where(pad, 0.0, 1.0)
    padf = jnp.where(pad, 1.0, 0.0)
    rep_r = lax.broadcasted_iota(jnp.int32, (NSAMPLE, SC), 0)
    rep_c = lax.broadcasted_iota(jnp.int32, (NSAMPLE, SC), 1)
    rep16 = jnp.where(rep_c // C_IN == rep_r, 1.0, 0.0).astype(jnp.bfloat16)
    r8r = lax.broadcasted_iota(jnp.int32, (NSAMPLE, SCW), 0)
    r8c = lax.broadcasted_iota(jnp.int32, (NSAMPLE, SCW), 1)
    rep8 = jnp.where(r8c // CW == r8r, 1.0, 0.0).astype(jnp.bfloat16)
    qr = lax.broadcasted_iota(jnp.int32, (CW, SCW), 0)
    qc = lax.broadcasted_iota(jnp.int32, (CW, SCW), 1)
    qrep = jnp.where(qc % CW == qr, 1.0, 0.0)
    sr = lax.broadcasted_iota(jnp.int32, (SCW, NSAMPLE), 0)
    scc = lax.broadcasted_iota(jnp.int32, (SCW, NSAMPLE), 1)
    seg = jnp.where(sr // CW == scc, 1.0, 0.0)
    qb = jnp.dot(q8[...], qrep, preferred_element_type=jnp.float32)
    pad128 = jnp.dot(padf.astype(jnp.bfloat16), rep8,
                     preferred_element_type=jnp.float32)
    cq = cg[...] - qb + pad128 * jnp.float32(PAD_XYZ)
    lm = lax.broadcasted_iota(jnp.int32, (1, SCW), 1) % CW
    f2v = f2[...]
    acc = jnp.zeros((RB, C_OUT), jnp.float32)
    for k in range(K):
        ax = kp_ref[k, 0]
        ay = kp_ref[k, 1]
        az = kp_ref[k, 2]
        kv = (jnp.where(lm == 0, ax, 0.0) + jnp.where(lm == 1, ay, 0.0)
              + jnp.where(lm == 2, az, 0.0))
        d = cq - kv
        sq = jnp.dot(d * d, seg, preferred_element_type=jnp.float32)
        awk = jnp.maximum(1.0 - jnp.sqrt(sq + 1e-9) / KP_EXTENT, 0.0)
        awb = jnp.dot(awk.astype(jnp.bfloat16), rep16,
                      preferred_element_type=jnp.float32)
        acc = acc + jnp.dot(awb.astype(jnp.bfloat16) * f2v, wrep_ref[k],
                            preferred_element_type=jnp.float32)
    out_ref[0] = acc.T
    keepb = jnp.dot(keep.astype(jnp.bfloat16), rep16,
                    preferred_element_type=jnp.float32)
    xim = xi2[...] * keepb.astype(jnp.bfloat16)
    m = xim[:, 0:C_IN]
    for s in range(1, NSAMPLE):
        m = jnp.maximum(m, xim[:, s * C_IN:(s + 1) * C_IN])
    skip_ref[0] = m.astype(jnp.float32).T


def _tc_stage(nidx, cg, q8, f2, xi2, wrep, kernel_points):
    grid = (B * N // RB,)
    return pl.pallas_call(
        _tc_body,
        grid=grid,
        in_specs=[
            pl.BlockSpec((RB, NSAMPLE), lambda i: (i, 0)),
            pl.BlockSpec((RB, SCW), lambda i: (i, 0)),
            pl.BlockSpec((RB, CW), lambda i: (i, 0)),
            pl.BlockSpec((RB, SC), lambda i: (i, 0)),
            pl.BlockSpec((RB, SC), lambda i: (i, 0)),
            pl.BlockSpec((K, SC, C_OUT), lambda i: (0, 0, 0)),
            pl.BlockSpec((K, 3), lambda i: (0, 0),
                         memory_space=pltpu.SMEM),
        ],
        out_specs=[
            pl.BlockSpec((1, C_OUT, RB), lambda i: (i // NBN, 0, i % NBN)),
            pl.BlockSpec((1, C_OUT, RB), lambda i: (i // NBN, 0, i % NBN)),
        ],
        out_shape=[
            jax.ShapeDtypeStruct((B, C_OUT, N), jnp.float32),
            jax.ShapeDtypeStruct((B, C_OUT, N), jnp.float32),
        ],
    )(nidx, cg, q8, f2, xi2, wrep, kernel_points)


def kernel(p, x, x_in, weights, kernel_points):
    px = p[:, :, 0].reshape(B * N)
    py = p[:, :, 1].reshape(B * N)
    pz = p[:, :, 2].reshape(B * N)
    xt = jnp.transpose(x, (0, 2, 1)).reshape(B * N, C_IN).astype(jnp.bfloat16)
    xit = jnp.transpose(x_in, (0, 2, 1)).reshape(B * N, C_IN).astype(jnp.bfloat16)
    ctab = jnp.concatenate(
        [p.reshape(B * N, 3), jnp.zeros((B * N, CW - 3), jnp.float32)],
        axis=1)
    nidx_f, feat, xing, coord = _sc_stage(px, py, pz, xt, xit, ctab)
    f2 = feat.reshape(B * N, SC)
    xi2 = xing.reshape(B * N, SC)
    cg = coord.reshape(B * N, SCW)
    wrep = jnp.tile(weights, (1, NSAMPLE, 1)).astype(jnp.bfloat16)
    out, skip = _tc_stage(nidx_f, cg, ctab, f2, xi2, wrep, kernel_points)
    return out, p, skip, nidx_f.reshape(B, N, NSAMPLE)


# speedup vs baseline: 1.2383x; 1.2383x over previous
import functools

import jax
import jax.numpy as jnp
from jax import lax
from jax.experimental import pallas as pl
from jax.experimental.pallas import tpu as pltpu
from jax.experimental.pallas import tpu_sc as plsc

B, N, C_IN, C_OUT = 2, 4096, 64, 64
K = 15
RADIUS = 2.5
NSAMPLE = 16
KP_EXTENT = 1.2

NSUB = 16
PTS_PER_SUB = N // NSUB
GRP = 8
NGRP = PTS_PER_SUB // GRP
NCHUNK = N // 16
PAD_XYZ = 1000000.0
SC = NSAMPLE * C_IN


_SC_MESH = plsc.VectorSubcoreMesh(core_axis_name="c", subcore_axis_name="s")


def _sc_body(px, py, pz, xt, xit,
             nidx_o, relx_o, rely_o, relz_o, feat_o, xing_o,
             pxv, pyv, pzv, idxbuf,
             st_nidx, st_relx, st_rely, st_relz,
             gidx0, gidx1, rows_x0, rows_xi0, rows_x1, rows_xi1,
             gsx0, gsxi0, gsx1, gsxi1, wsx0, wsxi0, wsx1, wsxi1):
    b = lax.axis_index("c")
    w = lax.axis_index("s")
    bN = b * N
    pltpu.sync_copy(px.at[pl.ds(bN, N)], pxv)
    pltpu.sync_copy(py.at[pl.ds(bN, N)], pyv)
    pltpu.sync_copy(pz.at[pl.ds(bN, N)], pzv)
    base_local = w * PTS_PER_SUB
    r2 = jnp.float32(RADIUS * RADIUS)
    lanes = lax.iota(jnp.int32, 16)

    def scan_group(g, gidx):
        def point_body(t, carry):
            i_loc = base_local + carry + t
            isplat = jnp.full((16,), i_loc, jnp.int32)
            qx = plsc.load_gather(pxv, [isplat])
            qy = plsc.load_gather(pyv, [isplat])
            qz = plsc.load_gather(pzv, [isplat])

            def scan_cond(st):
                j, cnt = st
                return jnp.logical_and(cnt < NSAMPLE, j < NCHUNK)

            def scan_body(st):
                j, cnt = st
                off = j * 16
                jv = lanes + off
                dx = pxv[pl.ds(off, 16)] - qx
                dy = pyv[pl.ds(off, 16)] - qy
                dz = pzv[pl.ds(off, 16)] - qz
                sq = dx * dx + dy * dy + dz * dz
                m = sq <= r2
                plsc.store_compressed(idxbuf.at[pl.ds(cnt, 16)], jv, mask=m)
                return j + 1, cnt + jnp.sum(m.astype(jnp.int32))

            _, cnt = lax.while_loop(scan_cond, scan_body,
                                    (jnp.int32(0), jnp.int32(0)))
            f = jnp.minimum(cnt, NSAMPLE)
            idx16 = idxbuf[pl.ds(0, 16)]
            first = idx16[0]
            valid = lanes < f
            idxv = jnp.where(valid, idx16, first)
            gx = plsc.load_gather(pxv, [idxv])
            gy = plsc.load_gather(pyv, [idxv])
            gz = plsc.load_gather(pzv, [idxv])
            big = jnp.float32(PAD_XYZ)
            st_nidx[t, :] = idxv
            st_relx[t, :] = jnp.where(valid, gx - qx, big)
            st_rely[t, :] = jnp.where(valid, gy - qy, big)
            st_relz[t, :] = jnp.where(valid, gz - qz, big)
            gidx[pl.ds(t * 16, 16)] = idxv + bN
            return carry

        lax.fori_loop(0, GRP, point_body, g * GRP)

    def emit_st(g):
        gbase = bN + base_local + g * GRP
        pltpu.sync_copy(st_nidx, nidx_o.at[pl.ds(gbase, GRP)])
        pltpu.sync_copy(st_relx, relx_o.at[pl.ds(gbase, GRP)])
        pltpu.sync_copy(st_rely, rely_o.at[pl.ds(gbase, GRP)])
        pltpu.sync_copy(st_relz, relz_o.at[pl.ds(gbase, GRP)])

    def fslice(g):
        gbase = bN + base_local + g * GRP
        return pl.ds(gbase * 16, GRP * 16)

    def issue_gather(gidx, rx, rxi, sx, sxi):
        pltpu.async_copy(xt.at[gidx], rx, sx)
        pltpu.async_copy(xit.at[gidx], rxi, sxi)

    def wait_gather(gidx, rx, rxi, sx, sxi):
        pltpu.make_async_copy(xt.at[gidx], rx, sx).wait()
        pltpu.make_async_copy(xit.at[gidx], rxi, sxi).wait()

    def issue_wb(g, rx, rxi, sx, sxi):
        pltpu.async_copy(rx, feat_o.at[fslice(g)], sx)
        pltpu.async_copy(rxi, xing_o.at[fslice(g)], sxi)

    def wait_wb(g, rx, rxi, sx, sxi):
        pltpu.make_async_copy(rx, feat_o.at[fslice(g)], sx).wait()
        pltpu.make_async_copy(rxi, xing_o.at[fslice(g)], sxi).wait()

    scan_group(0, gidx0)
    issue_gather(gidx0, rows_x0, rows_xi0, gsx0, gsxi0)
    emit_st(0)
    scan_group(1, gidx1)
    issue_gather(gidx1, rows_x1, rows_xi1, gsx1, gsxi1)
    emit_st(1)
    wait_gather(gidx0, rows_x0, rows_xi0, gsx0, gsxi0)
    issue_wb(0, rows_x0, rows_xi0, wsx0, wsxi0)

    def pair_body(h, _):
        g0 = 2 * h
        g1 = g0 + 1
        scan_group(g0, gidx0)
        wait_wb(g0 - 2, rows_x0, rows_xi0, wsx0, wsxi0)
        issue_gather(gidx0, rows_x0, rows_xi0, gsx0, gsxi0)
        emit_st(g0)
        wait_gather(gidx1, rows_x1, rows_xi1, gsx1, gsxi1)
        issue_wb(g0 - 1, rows_x1, rows_xi1, wsx1, wsxi1)
        scan_group(g1, gidx1)
        wait_wb(g0 - 1, rows_x1, rows_xi1, wsx1, wsxi1)
        issue_gather(gidx1, rows_x1, rows_xi1, gsx1, gsxi1)
        emit_st(g1)
        wait_gather(gidx0, rows_x0, rows_xi0, gsx0, gsxi0)
        issue_wb(g0, rows_x0, rows_xi0, wsx0, wsxi0)
        return 0

    lax.fori_loop(1, NGRP // 2, pair_body, 0)
    wait_gather(gidx1, rows_x1, rows_xi1, gsx1, gsxi1)
    pltpu.sync_copy(rows_x1, feat_o.at[fslice(NGRP - 1)])
    pltpu.sync_copy(rows_xi1, xing_o.at[fslice(NGRP - 1)])
    wait_wb(NGRP - 2, rows_x0, rows_xi0, wsx0, wsxi0)


_sc_stage = functools.partial(
    pl.kernel,
    out_type=(
        jax.ShapeDtypeStruct((B * N, NSAMPLE), jnp.int32),
        jax.ShapeDtypeStruct((B * N, NSAMPLE), jnp.float32),
        jax.ShapeDtypeStruct((B * N, NSAMPLE), jnp.float32),
        jax.ShapeDtypeStruct((B * N, NSAMPLE), jnp.float32),
        jax.ShapeDtypeStruct((B * N * NSAMPLE, C_IN), jnp.bfloat16),
        jax.ShapeDtypeStruct((B * N * NSAMPLE, C_IN), jnp.bfloat16),
    ),
    mesh=_SC_MESH,
    compiler_params=pltpu.CompilerParams(
        needs_layout_passes=False,
        use_tc_tiling_on_sc=False,
    ),
    scratch_types=[
        pltpu.VMEM((N,), jnp.float32),
        pltpu.VMEM((N,), jnp.float32),
        pltpu.VMEM((N,), jnp.float32),
        pltpu.VMEM((48,), jnp.int32),
        pltpu.VMEM((GRP, 16), jnp.int32),
        pltpu.VMEM((GRP, 16), jnp.float32),
        pltpu.VMEM((GRP, 16), jnp.float32),
        pltpu.VMEM((GRP, 16), jnp.float32),
        pltpu.VMEM((GRP * 16,), jnp.int32),
        pltpu.VMEM((GRP * 16,), jnp.int32),
        pltpu.VMEM((GRP * 16, C_IN), jnp.bfloat16),
        pltpu.VMEM((GRP * 16, C_IN), jnp.bfloat16),
        pltpu.VMEM((GRP * 16, C_IN), jnp.bfloat16),
        pltpu.VMEM((GRP * 16, C_IN), jnp.bfloat16),
        pltpu.SemaphoreType.DMA,
        pltpu.SemaphoreType.DMA,
        pltpu.SemaphoreType.DMA,
        pltpu.SemaphoreType.DMA,
        pltpu.SemaphoreType.DMA,
        pltpu.SemaphoreType.DMA,
        pltpu.SemaphoreType.DMA,
        pltpu.SemaphoreType.DMA,
    ],
)(_sc_body)


RB = 256
NBN = N // RB


def _tc_body(nidx, relx, rely, relz, f2, xi2, wrep_ref, kp_ref,
             out_ref, skip_ref):
    idx = nidx[...]
    s_iota = lax.broadcasted_iota(jnp.int32, (RB, NSAMPLE), 1)
    pad = jnp.logical_and(idx == idx[:, 0:1], s_iota > 0)
    keep = jnp.where(pad, 0.0, 1.0)
    rx = relx[...]
    ry = rely[...]
    rz = relz[...]
    rep_r = lax.broadcasted_iota(jnp.int32, (NSAMPLE, SC), 0)
    rep_c = lax.broadcasted_iota(jnp.int32, (NSAMPLE, SC), 1)
    rep16 = jnp.where(rep_c // C_IN == rep_r, 1.0, 0.0).astype(jnp.bfloat16)
    f2v = f2[...]
    acc = jnp.zeros((RB, C_OUT), jnp.float32)
    for k in range(K):
        ax = kp_ref[k, 0]
        ay = kp_ref[k, 1]
        az = kp_ref[k, 2]
        sq = (rx - ax) ** 2 + (ry - ay) ** 2 + (rz - az) ** 2
        awk = jnp.maximum(1.0 - jnp.sqrt(sq + 1e-9) / KP_EXTENT, 0.0)
        awb = jnp.dot(awk.astype(jnp.bfloat16), rep16,
                      preferred_element_type=jnp.float32)
        acc = acc + jnp.dot(awb.astype(jnp.bfloat16) * f2v, wrep_ref[k],
                            preferred_element_type=jnp.float32)
    out_ref[0] = acc.T
    keepb = jnp.dot(keep.astype(jnp.bfloat16), rep16,
                    preferred_element_type=jnp.float32)
    xim = xi2[...] * keepb.astype(jnp.bfloat16)
    m = xim[:, 0:C_IN]
    for s in range(1, NSAMPLE):
        m = jnp.maximum(m, xim[:, s * C_IN:(s + 1) * C_IN])
    skip_ref[0] = m.astype(jnp.float32).T


def _tc_stage(nidx, relx, rely, relz, f2, xi2, wrep, kernel_points):
    grid = (B * N // RB,)
    fspec = pl.BlockSpec((RB, NSAMPLE), lambda i: (i, 0))
    return pl.pallas_call(
        _tc_body,
        grid=grid,
        in_specs=[
            pl.BlockSpec((RB, NSAMPLE), lambda i: (i, 0)),
            fspec, fspec, fspec,
            pl.BlockSpec((RB, SC), lambda i: (i, 0)),
            pl.BlockSpec((RB, SC), lambda i: (i, 0)),
            pl.BlockSpec((K, SC, C_OUT), lambda i: (0, 0, 0)),
            pl.BlockSpec((K, 3), lambda i: (0, 0),
                         memory_space=pltpu.SMEM),
        ],
        out_specs=[
            pl.BlockSpec((1, C_OUT, RB), lambda i: (i // NBN, 0, i % NBN)),
            pl.BlockSpec((1, C_OUT, RB), lambda i: (i // NBN, 0, i % NBN)),
        ],
        out_shape=[
            jax.ShapeDtypeStruct((B, C_OUT, N), jnp.float32),
            jax.ShapeDtypeStruct((B, C_OUT, N), jnp.float32),
        ],
    )(nidx, relx, rely, relz, f2, xi2, wrep, kernel_points)


def kernel(p, x, x_in, weights, kernel_points):
    px = p[:, :, 0].reshape(B * N)
    py = p[:, :, 1].reshape(B * N)
    pz = p[:, :, 2].reshape(B * N)
    xt = jnp.transpose(x, (0, 2, 1)).reshape(B * N, C_IN).astype(jnp.bfloat16)
    xit = jnp.transpose(x_in, (0, 2, 1)).reshape(B * N, C_IN).astype(jnp.bfloat16)
    nidx_f, relx, rely, relz, feat, xing = _sc_stage(px, py, pz, xt, xit)
    f2 = feat.reshape(B * N, SC)
    xi2 = xing.reshape(B * N, SC)
    wrep = jnp.tile(weights, (1, NSAMPLE, 1)).astype(jnp.bfloat16)
    out, skip = _tc_stage(nidx_f, relx, rely, relz, f2, xi2,
                          wrep, kernel_points)
    return out, p, skip, nidx_f.reshape(B, N, NSAMPLE)
